# baseline (speedup 1.0000x reference)
import jax
import jax.numpy as jnp
from jax import lax
from jax.experimental import pallas as pl
from jax.experimental.pallas import tpu as pltpu
from jax.experimental.pallas import tpu_sc as plsc

VOCAB = 100000
DIM = 128
NUM_TOKENS = 20
BATCH = 1024
SEQ = 220
CHUNK = 256
CPP = BATCH // CHUNK
NCHUNKS = SEQ * CPP
NBUF = 3

_info = plsc.get_sparse_core_info()
_NC, _NS = _info.num_cores, _info.num_subcores
NW = _NC * _NS
STEPS = -(-NCHUNKS // NW)


def _sc_body(tok_hbm, table_hbm, prompt_hbm, out_hbm,
             idx0, idx1, idx2, buf0, buf1, buf2,
             sem_g0, sem_g1, sem_g2, sem_w0, sem_w1, sem_w2):
    wid = lax.axis_index("s") * _NC + lax.axis_index("c")

    idxs = (idx0, idx1, idx2)
    bufs = (buf0, buf1, buf2)
    sems_g = (sem_g0, sem_g1, sem_g2)
    sems_w = (sem_w0, sem_w1, sem_w2)

    def issue(s, k):
        t = wid + s * NW
        t = jnp.where(t < NCHUNKS, t, wid)
        j = t // CPP
        c = t % CPP
        is_prompt = j < NUM_TOKENS

        @pl.when(is_prompt)
        def _():
            fill = jnp.broadcast_to(j.astype(jnp.int32), (16,))
            for q in range(CHUNK // 16):
                idxs[k][pl.ds(q * 16, 16)] = fill
            pltpu.async_copy(prompt_hbm.at[idxs[k]], bufs[k], sems_g[k])

        @pl.when(jnp.logical_not(is_prompt))
        def _():
            pltpu.sync_copy(tok_hbm.at[pl.ds(j * BATCH + c * CHUNK, CHUNK)],
                            idxs[k])
            pltpu.async_copy(table_hbm.at[idxs[k]], bufs[k], sems_g[k])

        return (j, c)

    def wait_gather(k):
        pltpu.make_async_copy(table_hbm.at[idxs[k]], bufs[k], sems_g[k]).wait()

    pend_jc = [None] * NBUF
    pend_w = [None] * NBUF
    for s in range(NBUF - 1):
        pend_jc[s] = issue(s, s)
    for s in range(STEPS):
        k = s % NBUF
        nk = (s + NBUF - 1) % NBUF
        if s + NBUF - 1 < STEPS:
            if pend_w[nk] is not None:
                pend_w[nk].wait()
                pend_w[nk] = None
            pend_jc[nk] = issue(s + NBUF - 1, nk)
        wait_gather(k)
        j, c = pend_jc[k]
        pend_w[k] = pltpu.async_copy(bufs[k], out_hbm.at[j, pl.ds(c * CHUNK, CHUNK)],
                                     sems_w[k])
    for k in range(NBUF):
        if pend_w[k] is not None:
            pend_w[k].wait()


def kernel(tokens, table, prompt_embedding):
    tok = jnp.transpose(tokens.astype(jnp.int32)).reshape(-1)
    sc = pl.kernel(
        _sc_body,
        out_type=jax.ShapeDtypeStruct((SEQ, BATCH, DIM), jnp.float32),
        mesh=plsc.VectorSubcoreMesh(core_axis_name="c", subcore_axis_name="s"),
        scratch_types=[
            pltpu.VMEM((CHUNK,), jnp.int32),
            pltpu.VMEM((CHUNK,), jnp.int32),
            pltpu.VMEM((CHUNK,), jnp.int32),
            pltpu.VMEM((CHUNK, DIM), jnp.float32),
            pltpu.VMEM((CHUNK, DIM), jnp.float32),
            pltpu.VMEM((CHUNK, DIM), jnp.float32),
            pltpu.SemaphoreType.DMA,
            pltpu.SemaphoreType.DMA,
            pltpu.SemaphoreType.DMA,
            pltpu.SemaphoreType.DMA,
            pltpu.SemaphoreType.DMA,
            pltpu.SemaphoreType.DMA,
        ],
    )
    out = sc(tok, table, prompt_embedding)
    return jnp.transpose(out, (1, 0, 2))
